# baseline (speedup 1.0000x reference)
import functools

import jax
import jax.numpy as jnp
from jax import lax
from jax.experimental import pallas as pl
from jax.experimental.pallas import tpu as pltpu
from jax.experimental.pallas import tpu_sc as plsc

_BATCH = 16384
_UN = 1000000
_IN = 100000

_NS = 16
_CHUNK = 128
_BW = _BATCH // _NS
_NCHUNK = _BW // _CHUNK

_UPW = (_UN // 128 // _NS) * 128
_UTAIL0 = _UPW * _NS
_UFULL = (_UN // 128) * 128
_IPW = (_IN // 128 // _NS) * 128
_ITAIL0 = _IPW * _NS
_IFULL = (_IN // 128) * 128


def _mf_body(x_hbm, y_hbm, ut_hbm, it_hbm, ut0t, ut1t, it0t, it1t,
             out_hbm, u0s, u1s, i0s, i1s,
             xv, yv, u0b, u1b, i0b, i1b, accv, allv, outv,
             usem, vsem):
    wid = lax.axis_index("s")

    for (src, d0, d1, pw, tail0, full, t0, t1) in (
        (ut_hbm, u0s, u1s, _UPW, _UTAIL0, _UFULL, ut0t, ut1t),
        (it_hbm, i0s, i1s, _IPW, _ITAIL0, _IFULL, it0t, it1t),
    ):
        sl = pl.ds(wid * pw, pw)
        pltpu.sync_copy(src.at[0].at[sl], d0.at[sl])
        pltpu.sync_copy(src.at[1].at[sl], d1.at[sl])

        @pl.when(wid == _NS - 1)
        def _():
            tl = pl.ds(tail0, full - tail0)
            pltpu.sync_copy(src.at[0].at[tl], d0.at[tl])
            pltpu.sync_copy(src.at[1].at[tl], d1.at[tl])
            pltpu.sync_copy(t0, d0.at[pl.ds(full, 128)])
            pltpu.sync_copy(t1, d1.at[pl.ds(full, 128)])

    pltpu.sync_copy(x_hbm.at[pl.ds(wid * _BW, _BW)], xv)
    pltpu.sync_copy(y_hbm.at[pl.ds(wid * _BW, _BW)], yv)

    plsc.subcore_barrier()

    copies = []
    for c in range(_NCHUNK):
        xi = xv.at[pl.ds(c * _CHUNK, _CHUNK)]
        yi = yv.at[pl.ds(c * _CHUNK, _CHUNK)]
        copies.append(pltpu.async_copy(u0s.at[xi], u0b.at[c], usem))
        copies.append(pltpu.async_copy(u1s.at[xi], u1b.at[c], usem))
        copies.append(pltpu.async_copy(i0s.at[yi], i0b.at[c], vsem))
        copies.append(pltpu.async_copy(i1s.at[yi], i1b.at[c], vsem))
    for cp in copies:
        cp.wait()

    acc = jnp.zeros((16,), jnp.float32)
    for c in range(_NCHUNK):
        for k in range(_CHUNK // 16):
            sl = pl.ds(16 * k, 16)
            acc = acc + u0b[c, sl] * i0b[c, sl] + u1b[c, sl] * i1b[c, sl]

    accv[...] = acc
    pltpu.sync_copy(accv, out_hbm.at[pl.ds(wid * 16, 16)])
    plsc.subcore_barrier()

    @pl.when(wid == 0)
    def _():
        pltpu.sync_copy(out_hbm.at[pl.ds(0, _NS * 16)], allv)
        tot = jnp.zeros((16,), jnp.float32)
        for r in range(_NS):
            tot = tot + allv[pl.ds(16 * r, 16)]
        s = tot[0]
        for l in range(1, 16):
            s = s + tot[l]
        outv[...] = jnp.broadcast_to(s, (16,))
        pltpu.sync_copy(outv, out_hbm.at[pl.ds(_NS * 16, 16)])


@functools.partial(
    pl.kernel,
    mesh=plsc.VectorSubcoreMesh(core_axis_name="c", subcore_axis_name="s",
                                num_cores=1),
    out_type=(
        jax.ShapeDtypeStruct(((_NS + 1) * 16,), jnp.float32),
        jax.ShapeDtypeStruct((_UFULL + 128,), jnp.float32),
        jax.ShapeDtypeStruct((_UFULL + 128,), jnp.float32),
        jax.ShapeDtypeStruct((_IFULL + 128,), jnp.float32),
        jax.ShapeDtypeStruct((_IFULL + 128,), jnp.float32),
    ),
    scratch_types=[
        pltpu.VMEM((_BW,), jnp.int32),
        pltpu.VMEM((_BW,), jnp.int32),
        pltpu.VMEM((_NCHUNK, _CHUNK), jnp.float32),
        pltpu.VMEM((_NCHUNK, _CHUNK), jnp.float32),
        pltpu.VMEM((_NCHUNK, _CHUNK), jnp.float32),
        pltpu.VMEM((_NCHUNK, _CHUNK), jnp.float32),
        pltpu.VMEM((16,), jnp.float32),
        pltpu.VMEM((_NS * 16,), jnp.float32),
        pltpu.VMEM((16,), jnp.float32),
        pltpu.SemaphoreType.DMA,
        pltpu.SemaphoreType.DMA,
    ],
)
def _mf(x_hbm, y_hbm, ut_hbm, it_hbm, ut0t, ut1t, it0t, it1t,
        out_hbm, u0s, u1s, i0s, i1s, *scratch):
    _mf_body(x_hbm, y_hbm, ut_hbm, it_hbm, ut0t, ut1t, it0t, it1t,
             out_hbm, u0s, u1s, i0s, i1s, *scratch)


def kernel(x, y, user_table, item_table):
    ut_tail = user_table[_UFULL:]
    it_tail = item_table[_IFULL:]
    outs = _mf(x, y, user_table.T, item_table.T,
               jnp.pad(ut_tail[:, 0], (0, 128 - (_UN - _UFULL))),
               jnp.pad(ut_tail[:, 1], (0, 128 - (_UN - _UFULL))),
               jnp.pad(it_tail[:, 0], (0, 128 - (_IN - _IFULL))),
               jnp.pad(it_tail[:, 1], (0, 128 - (_IN - _IFULL))))
    return outs[0][_NS * 16]


# speedup vs baseline: 4.0460x; 4.0460x over previous
import functools

import jax
import jax.numpy as jnp
from jax import lax
from jax.experimental import pallas as pl
from jax.experimental.pallas import tpu as pltpu
from jax.experimental.pallas import tpu_sc as plsc

_BATCH = 16384

_NS = 16
_CHUNK = 128
_BW = _BATCH // _NS
_NCHUNK = _BW // _CHUNK


def _mf_body(x_hbm, y_hbm, u0_hbm, u1_hbm, i0_hbm, i1_hbm, out_hbm,
             xv, yv, u0b, u1b, i0b, i1b, accv, allv, outv,
             usem, vsem):
    wid = lax.axis_index("s")

    pltpu.sync_copy(x_hbm.at[pl.ds(wid * _BW, _BW)], xv)
    pltpu.sync_copy(y_hbm.at[pl.ds(wid * _BW, _BW)], yv)

    copies = []
    for c in range(_NCHUNK):
        xi = xv.at[pl.ds(c * _CHUNK, _CHUNK)]
        yi = yv.at[pl.ds(c * _CHUNK, _CHUNK)]
        copies.append(pltpu.async_copy(u0_hbm.at[xi], u0b.at[c], usem))
        copies.append(pltpu.async_copy(u1_hbm.at[xi], u1b.at[c], usem))
        copies.append(pltpu.async_copy(i0_hbm.at[yi], i0b.at[c], vsem))
        copies.append(pltpu.async_copy(i1_hbm.at[yi], i1b.at[c], vsem))
    for cp in copies:
        cp.wait()

    acc = jnp.zeros((16,), jnp.float32)
    for c in range(_NCHUNK):
        for k in range(_CHUNK // 16):
            sl = pl.ds(16 * k, 16)
            acc = acc + u0b[c, sl] * i0b[c, sl] + u1b[c, sl] * i1b[c, sl]

    accv[...] = acc
    pltpu.sync_copy(accv, out_hbm.at[pl.ds(wid * 16, 16)])
    plsc.subcore_barrier()

    @pl.when(wid == 0)
    def _():
        pltpu.sync_copy(out_hbm.at[pl.ds(0, _NS * 16)], allv)
        tot = jnp.zeros((16,), jnp.float32)
        for r in range(_NS):
            tot = tot + allv[pl.ds(16 * r, 16)]
        s = tot[0]
        for l in range(1, 16):
            s = s + tot[l]
        outv[...] = jnp.broadcast_to(s, (16,))
        pltpu.sync_copy(outv, out_hbm.at[pl.ds(_NS * 16, 16)])


@functools.partial(
    pl.kernel,
    mesh=plsc.VectorSubcoreMesh(core_axis_name="c", subcore_axis_name="s",
                                num_cores=1),
    out_type=jax.ShapeDtypeStruct(((_NS + 1) * 16,), jnp.float32),
    scratch_types=[
        pltpu.VMEM((_BW,), jnp.int32),
        pltpu.VMEM((_BW,), jnp.int32),
        pltpu.VMEM((_NCHUNK, _CHUNK), jnp.float32),
        pltpu.VMEM((_NCHUNK, _CHUNK), jnp.float32),
        pltpu.VMEM((_NCHUNK, _CHUNK), jnp.float32),
        pltpu.VMEM((_NCHUNK, _CHUNK), jnp.float32),
        pltpu.VMEM((16,), jnp.float32),
        pltpu.VMEM((_NS * 16,), jnp.float32),
        pltpu.VMEM((16,), jnp.float32),
        pltpu.SemaphoreType.DMA,
        pltpu.SemaphoreType.DMA,
    ],
)
def _mf(x_hbm, y_hbm, u0_hbm, u1_hbm, i0_hbm, i1_hbm, out_hbm, *scratch):
    _mf_body(x_hbm, y_hbm, u0_hbm, u1_hbm, i0_hbm, i1_hbm, out_hbm, *scratch)


def kernel(x, y, user_table, item_table):
    out = _mf(x, y,
              user_table[:, 0], user_table[:, 1],
              item_table[:, 0], item_table[:, 1])
    return out[_NS * 16]
